# Initial kernel scaffold; baseline (speedup 1.0000x reference)
import functools

import jax
import jax.numpy as jnp
from jax import lax
from jax.experimental import pallas as pl
from jax.experimental.pallas import tpu as pltpu
from jax.experimental.pallas import tpu_sc as plsc

N = 10000
E = 320000
D = 128

NC = 2
NS = 16
LANES = 16
NW = NC * NS

CHUNK = 128
CPT = 79
EP = NW * CPT * CHUNK
NPAD = 10240
RPT = NPAD // NS
PAD_IDX = N + 16

_sc_mesh = plsc.VectorSubcoreMesh(core_axis_name="c", subcore_axis_name="s")


def _deg_body(col_hbm, ew_hbm, degp_hbm, col_v, ew_v, zb_v, acc_sh):
    c = lax.axis_index("c")
    s = lax.axis_index("s")
    w = c * NS + s

    pltpu.sync_copy(col_hbm.at[pl.ds(w * CPT, CPT)], col_v)
    pltpu.sync_copy(ew_hbm.at[pl.ds(w * CPT, CPT)], ew_v)

    def zero(i, carry):
        zb_v[pl.ds(i * LANES, LANES)] = jnp.zeros((LANES,), jnp.float32)
        return carry

    lax.fori_loop(0, RPT // LANES, zero, 0)
    pltpu.sync_copy(zb_v, acc_sh.at[pl.ds(s * RPT, RPT)])
    plsc.subcore_barrier()

    def body(j, carry):
        pltpu.sync_copy(ew_v.at[j], acc_sh.at[col_v.at[j]], add=True)
        return carry

    lax.fori_loop(0, CPT, body, 0)
    plsc.subcore_barrier()
    pltpu.sync_copy(
        acc_sh.at[pl.ds(s * RPT, RPT)],
        degp_hbm.at[c, pl.ds(s * RPT, RPT)],
    )


_deg_call = pl.kernel(
    _deg_body,
    out_type=jax.ShapeDtypeStruct((NC, NPAD), jnp.float32),
    mesh=_sc_mesh,
    scratch_types=[
        pltpu.VMEM((CPT, CHUNK), jnp.int32),
        pltpu.VMEM((CPT, CHUNK), jnp.float32),
        pltpu.VMEM((RPT,), jnp.float32),
        pltpu.VMEM_SHARED((NPAD,), jnp.float32),
    ],
)


ZB = 64


def _msg_body(row_hbm, col_hbm, ew_hbm, dis_hbm, h_hbm, accp_hbm,
              row_v, col_v, ew_v, dis_v, g_v, rows_v, zb_v, acc_sh, sem):
    c = lax.axis_index("c")
    s = lax.axis_index("s")
    w = c * NS + s

    pltpu.sync_copy(row_hbm.at[pl.ds(w * CPT, CPT)], row_v)
    pltpu.sync_copy(col_hbm.at[pl.ds(w * CPT, CPT)], col_v)
    pltpu.sync_copy(ew_hbm.at[pl.ds(w * CPT, CPT)], ew_v)
    pltpu.sync_copy(dis_hbm, dis_v)

    def zero(i, carry):
        for v in range(D // LANES):
            zb_v[i, pl.ds(v * LANES, LANES)] = jnp.zeros(
                (LANES,), jnp.float32)
        return carry

    lax.fori_loop(0, ZB, zero, 0)

    def zcopy(k, carry):
        pltpu.sync_copy(zb_v, acc_sh.at[pl.ds(s * RPT + k * ZB, ZB)])
        return carry

    lax.fori_loop(0, RPT // ZB, zcopy, 0)
    plsc.subcore_barrier()

    def chunk(j, carry):
        for v in range(CHUNK // LANES):
            idx16 = row_v[j, pl.ds(v * LANES, LANES)]
            d16 = plsc.load_gather(dis_v, [idx16])
            g_v[pl.ds(v * LANES, LANES)] = (
                d16 * ew_v[j, pl.ds(v * LANES, LANES)])

        pltpu.async_copy(h_hbm.at[row_v.at[j]], rows_v, sem).wait()

        def scale(e, carry2):
            gs = plsc.load_gather(g_v, [jnp.full((LANES,), e, jnp.int32)])
            for v in range(D // LANES):
                sl = pl.ds(v * LANES, LANES)
                rows_v[e, sl] = rows_v[e, sl] * gs
            return carry2

        lax.fori_loop(0, CHUNK, scale, 0)

        pltpu.sync_copy(rows_v, acc_sh.at[col_v.at[j]], add=True)
        return carry

    lax.fori_loop(0, CPT, chunk, 0)
    plsc.subcore_barrier()
    pltpu.sync_copy(
        acc_sh.at[pl.ds(s * RPT, RPT)],
        accp_hbm.at[c, pl.ds(s * RPT, RPT)],
    )


_msg_call = pl.kernel(
    _msg_body,
    out_type=jax.ShapeDtypeStruct((NC, NPAD, D), jnp.float32),
    mesh=_sc_mesh,
    scratch_types=[
        pltpu.VMEM((CPT, CHUNK), jnp.int32),
        pltpu.VMEM((CPT, CHUNK), jnp.int32),
        pltpu.VMEM((CPT, CHUNK), jnp.float32),
        pltpu.VMEM((NPAD,), jnp.float32),
        pltpu.VMEM((CHUNK,), jnp.float32),
        pltpu.VMEM((CHUNK, D), jnp.float32),
        pltpu.VMEM((ZB, D), jnp.float32),
        pltpu.VMEM_SHARED((NPAD, D), jnp.float32),
        pltpu.SemaphoreType.DMA,
    ],
)


MM_BLK = 1280


def _mm_body(x_ref, w_ref, o_ref):
    o_ref[...] = jnp.dot(x_ref[...], w_ref[...],
                         preferred_element_type=jnp.float32)


_mm_call = pl.pallas_call(
    _mm_body,
    grid=(NPAD // MM_BLK,),
    in_specs=[
        pl.BlockSpec((MM_BLK, D), lambda i: (i, 0)),
        pl.BlockSpec((D, D), lambda i: (0, 0)),
    ],
    out_specs=pl.BlockSpec((MM_BLK, D), lambda i: (i, 0)),
    out_shape=jax.ShapeDtypeStruct((NPAD, D), jnp.float32),
)


def _dis_body(degp_ref, dis_ref):
    deg = degp_ref[0, :] + degp_ref[1, :]
    safe = jnp.where(deg > 0, deg, 1.0)
    dis_ref[...] = jnp.where(deg > 0, lax.rsqrt(safe), 0.0)


_dis_call = pl.pallas_call(
    _dis_body,
    out_shape=jax.ShapeDtypeStruct((NPAD,), jnp.float32),
)


FIN_BLK = 1280


def _fin_body(accp_ref, dis_ref, b_ref, o_ref):
    acc = accp_ref[0] + accp_ref[1]
    o = acc * dis_ref[...] + b_ref[...]
    m = jnp.max(o, axis=1, keepdims=True)
    e = jnp.exp(o - m)
    o_ref[...] = e / jnp.sum(e, axis=1, keepdims=True)


_fin_call = pl.pallas_call(
    _fin_body,
    grid=(NPAD // FIN_BLK,),
    in_specs=[
        pl.BlockSpec((NC, FIN_BLK, D), lambda i: (0, i, 0)),
        pl.BlockSpec((FIN_BLK, 1), lambda i: (i, 0)),
        pl.BlockSpec((1, D), lambda i: (0, 0)),
    ],
    out_specs=pl.BlockSpec((FIN_BLK, D), lambda i: (i, 0)),
    out_shape=jax.ShapeDtypeStruct((NPAD, D), jnp.float32),
)


def kernel(x, edge_index, edge_weight, W, b):
    row = edge_index[0].astype(jnp.int32)
    col = edge_index[1].astype(jnp.int32)
    pad = EP - E
    rowp = jnp.concatenate(
        [row, jnp.full((pad,), PAD_IDX, jnp.int32)]).reshape(EP // CHUNK,
                                                             CHUNK)
    colp = jnp.concatenate(
        [col, jnp.full((pad,), PAD_IDX, jnp.int32)]).reshape(EP // CHUNK,
                                                             CHUNK)
    ewp = jnp.concatenate(
        [edge_weight.astype(jnp.float32),
         jnp.zeros((pad,), jnp.float32)]).reshape(EP // CHUNK, CHUNK)
    xp = jnp.concatenate(
        [x.astype(jnp.float32), jnp.zeros((NPAD - N, D), jnp.float32)])

    degp = _deg_call(colp, ewp)
    h = _mm_call(xp, W.astype(jnp.float32))
    dis = _dis_call(degp)
    accp = _msg_call(rowp, colp, ewp, dis, h)
    out = _fin_call(accp, dis.reshape(NPAD, 1), b.astype(jnp.float32))
    return out[:N]


# speedup vs baseline: 9.4685x; 9.4685x over previous
import functools

import jax
import jax.numpy as jnp
from jax import lax
from jax.experimental import pallas as pl
from jax.experimental.pallas import tpu as pltpu
from jax.experimental.pallas import tpu_sc as plsc

N = 10000
E = 320000
D = 128

NC = 2
NS = 16
LANES = 16
NW = NC * NS

CHUNK = 128
CPT = 80
EP = NW * CPT * CHUNK
NPAD = 10240
RPT = NPAD // NS
PAD_IDX = N + 16

_sc_mesh = plsc.VectorSubcoreMesh(core_axis_name="c", subcore_axis_name="s")


def _deg_body(col_hbm, ew_hbm, degp_hbm, col_v, ew_v, zb_v, acc_sh):
    c = lax.axis_index("c")
    s = lax.axis_index("s")
    w = c * NS + s

    pltpu.sync_copy(col_hbm.at[pl.ds(w * CPT, CPT)], col_v)
    pltpu.sync_copy(ew_hbm.at[pl.ds(w * CPT, CPT)], ew_v)

    def zero(i, carry):
        zb_v[pl.ds(i * LANES, LANES)] = jnp.zeros((LANES,), jnp.float32)
        return carry

    lax.fori_loop(0, RPT // LANES, zero, 0)
    pltpu.sync_copy(zb_v, acc_sh.at[pl.ds(s * RPT, RPT)])
    plsc.subcore_barrier()

    def body(j, carry):
        pltpu.sync_copy(ew_v.at[j], acc_sh.at[col_v.at[j]], add=True)
        return carry

    lax.fori_loop(0, CPT, body, 0)
    plsc.subcore_barrier()
    pltpu.sync_copy(
        acc_sh.at[pl.ds(s * RPT, RPT)],
        degp_hbm.at[c, pl.ds(s * RPT, RPT)],
    )


_sc_params = pltpu.CompilerParams(needs_layout_passes=False)

_deg_call = pl.kernel(
    _deg_body,
    out_type=jax.ShapeDtypeStruct((NC, NPAD), jnp.float32),
    mesh=_sc_mesh,
    compiler_params=_sc_params,
    scratch_types=[
        pltpu.VMEM((CPT, CHUNK), jnp.int32),
        pltpu.VMEM((CPT, CHUNK), jnp.float32),
        pltpu.VMEM((RPT,), jnp.float32),
        pltpu.VMEM_SHARED((NPAD,), jnp.float32),
    ],
)


ZB = 32
BB = 8


def _msg_body(row_hbm, col_hbm, ew_hbm, dis_hbm, h_hbm, accp_hbm,
              row_v, col_v, ew_v, dis_v, g_v, rows_v, zb_v, acc_sh, sem):
    c = lax.axis_index("c")
    s = lax.axis_index("s")
    w = c * NS + s

    pltpu.sync_copy(dis_hbm, dis_v)

    def zero(i, carry):
        for v in range(D // LANES):
            zb_v[i, pl.ds(v * LANES, LANES)] = jnp.zeros(
                (LANES,), jnp.float32)
        return carry

    lax.fori_loop(0, ZB, zero, 0)

    def zcopy(k, carry):
        pltpu.sync_copy(zb_v, acc_sh.at[pl.ds(s * RPT + k * ZB, ZB)])
        return carry

    lax.fori_loop(0, RPT // ZB, zcopy, 0)
    plsc.subcore_barrier()

    def block(bb, carry):
        base = w * CPT + bb * BB
        pltpu.sync_copy(row_hbm.at[pl.ds(base, BB)], row_v)
        pltpu.sync_copy(col_hbm.at[pl.ds(base, BB)], col_v)
        pltpu.sync_copy(ew_hbm.at[pl.ds(base, BB)], ew_v)

        def chunk(j, carry1):
            for v in range(CHUNK // LANES):
                idx16 = row_v[j, pl.ds(v * LANES, LANES)]
                d16 = plsc.load_gather(dis_v, [idx16])
                g_v[pl.ds(v * LANES, LANES)] = (
                    d16 * ew_v[j, pl.ds(v * LANES, LANES)])

            pltpu.async_copy(h_hbm.at[row_v.at[j]], rows_v, sem).wait()

            def scale(e, carry2):
                gs = plsc.load_gather(
                    g_v, [jnp.full((LANES,), e, jnp.int32)])
                for v in range(D // LANES):
                    sl = pl.ds(v * LANES, LANES)
                    rows_v[e, sl] = rows_v[e, sl] * gs
                return carry2

            lax.fori_loop(0, CHUNK, scale, 0)

            pltpu.sync_copy(rows_v, acc_sh.at[col_v.at[j]], add=True)
            return carry1

        lax.fori_loop(0, BB, chunk, 0)
        return carry

    lax.fori_loop(0, CPT // BB, block, 0)
    plsc.subcore_barrier()
    pltpu.sync_copy(
        acc_sh.at[pl.ds(s * RPT, RPT)],
        accp_hbm.at[c, pl.ds(s * RPT, RPT)],
    )


_msg_call = pl.kernel(
    _msg_body,
    out_type=jax.ShapeDtypeStruct((NC, NPAD, D), jnp.float32),
    mesh=_sc_mesh,
    compiler_params=_sc_params,
    scratch_types=[
        pltpu.VMEM((BB, CHUNK), jnp.int32),
        pltpu.VMEM((BB, CHUNK), jnp.int32),
        pltpu.VMEM((BB, CHUNK), jnp.float32),
        pltpu.VMEM((NPAD,), jnp.float32),
        pltpu.VMEM((CHUNK,), jnp.float32),
        pltpu.VMEM((CHUNK, D), jnp.float32),
        pltpu.VMEM((ZB, D), jnp.float32),
        pltpu.VMEM_SHARED((NPAD, D), jnp.float32),
        pltpu.SemaphoreType.DMA,
    ],
)


MM_BLK = 1280


def _mm_body(x_ref, w_ref, o_ref):
    o_ref[...] = jnp.dot(x_ref[...], w_ref[...],
                         preferred_element_type=jnp.float32)


_mm_call = pl.pallas_call(
    _mm_body,
    grid=(NPAD // MM_BLK,),
    in_specs=[
        pl.BlockSpec((MM_BLK, D), lambda i: (i, 0)),
        pl.BlockSpec((D, D), lambda i: (0, 0)),
    ],
    out_specs=pl.BlockSpec((MM_BLK, D), lambda i: (i, 0)),
    out_shape=jax.ShapeDtypeStruct((NPAD, D), jnp.float32),
)


def _dis_body(degp_ref, dis_ref):
    deg = degp_ref[0, :] + degp_ref[1, :]
    safe = jnp.where(deg > 0, deg, 1.0)
    dis_ref[...] = jnp.where(deg > 0, lax.rsqrt(safe), 0.0)


_dis_call = pl.pallas_call(
    _dis_body,
    out_shape=jax.ShapeDtypeStruct((NPAD,), jnp.float32),
)


FIN_BLK = 1280


def _fin_body(accp_ref, dis_ref, b_ref, o_ref):
    acc = accp_ref[0] + accp_ref[1]
    o = acc * dis_ref[...] + b_ref[...]
    m = jnp.max(o, axis=1, keepdims=True)
    e = jnp.exp(o - m)
    o_ref[...] = e / jnp.sum(e, axis=1, keepdims=True)


_fin_call = pl.pallas_call(
    _fin_body,
    grid=(NPAD // FIN_BLK,),
    in_specs=[
        pl.BlockSpec((NC, FIN_BLK, D), lambda i: (0, i, 0)),
        pl.BlockSpec((FIN_BLK, 1), lambda i: (i, 0)),
        pl.BlockSpec((1, D), lambda i: (0, 0)),
    ],
    out_specs=pl.BlockSpec((FIN_BLK, D), lambda i: (i, 0)),
    out_shape=jax.ShapeDtypeStruct((NPAD, D), jnp.float32),
)


def kernel(x, edge_index, edge_weight, W, b):
    row = edge_index[0].astype(jnp.int32)
    col = edge_index[1].astype(jnp.int32)
    pad = EP - E
    rowp = jnp.concatenate(
        [row, jnp.full((pad,), PAD_IDX, jnp.int32)]).reshape(EP // CHUNK,
                                                             CHUNK)
    colp = jnp.concatenate(
        [col, jnp.full((pad,), PAD_IDX, jnp.int32)]).reshape(EP // CHUNK,
                                                             CHUNK)
    ewp = jnp.concatenate(
        [edge_weight.astype(jnp.float32),
         jnp.zeros((pad,), jnp.float32)]).reshape(EP // CHUNK, CHUNK)
    xp = jnp.concatenate(
        [x.astype(jnp.float32), jnp.zeros((NPAD - N, D), jnp.float32)])

    degp = _deg_call(colp, ewp)
    h = _mm_call(xp, W.astype(jnp.float32))
    dis = _dis_call(degp)
    accp = _msg_call(rowp, colp, ewp, dis, h)
    out = _fin_call(accp, dis.reshape(NPAD, 1),
                    b.astype(jnp.float32).reshape(1, D))
    return out[:N]


# speedup vs baseline: 9.6364x; 1.0177x over previous
import functools

import jax
import jax.numpy as jnp
import numpy as np
from jax import lax
from jax.experimental import pallas as pl
from jax.experimental.pallas import tpu as pltpu
from jax.experimental.pallas import tpu_sc as plsc

N = 10000
E = 320000
D = 128

NC = 2
NS = 16
LANES = 16
NW = NC * NS

CHUNK = 128
CPT = 80
EP = NW * CPT * CHUNK
NPAD = 10240
RPT = NPAD // NS
PAD_IDX = N + 16

_sc_mesh = plsc.VectorSubcoreMesh(core_axis_name="c", subcore_axis_name="s")


def _deg_body(col_hbm, ew_hbm, degp_hbm, col_v, ew_v, zb_v, acc_sh):
    c = lax.axis_index("c")
    s = lax.axis_index("s")
    w = c * NS + s

    pltpu.sync_copy(col_hbm.at[pl.ds(w * CPT, CPT)], col_v)
    pltpu.sync_copy(ew_hbm.at[pl.ds(w * CPT, CPT)], ew_v)

    def zero(i, carry):
        zb_v[pl.ds(i * LANES, LANES)] = jnp.zeros((LANES,), jnp.float32)
        return carry

    lax.fori_loop(0, RPT // LANES, zero, 0)
    pltpu.sync_copy(zb_v, acc_sh.at[pl.ds(s * RPT, RPT)])
    plsc.subcore_barrier()

    def body(j, carry):
        pltpu.sync_copy(ew_v.at[j], acc_sh.at[col_v.at[j]], add=True)
        return carry

    lax.fori_loop(0, CPT, body, 0)
    plsc.subcore_barrier()
    pltpu.sync_copy(
        acc_sh.at[pl.ds(s * RPT, RPT)],
        degp_hbm.at[c, pl.ds(s * RPT, RPT)],
    )


_sc_params = pltpu.CompilerParams(needs_layout_passes=False)

_deg_call = pl.kernel(
    _deg_body,
    out_type=jax.ShapeDtypeStruct((NC, NPAD), jnp.float32),
    mesh=_sc_mesh,
    compiler_params=_sc_params,
    scratch_types=[
        pltpu.VMEM((CPT, CHUNK), jnp.int32),
        pltpu.VMEM((CPT, CHUNK), jnp.float32),
        pltpu.VMEM((RPT,), jnp.float32),
        pltpu.VMEM_SHARED((NPAD,), jnp.float32),
    ],
)


BB = 8


def _msg_body(row_hbm, col_hbm, ew_hbm, dis_hbm, h_hbm, accp_hbm,
              row_v, col_v, ew_v, dis_v, g_v, rows_a, rows_b, acc_sh,
              gsem_a, gsem_b):
    c = lax.axis_index("c")
    s = lax.axis_index("s")
    w = c * NS + s

    pltpu.sync_copy(dis_hbm, dis_v)

    def zrow(i, carry):
        for v in range(D // LANES):
            rows_a[i, pl.ds(v * LANES, LANES)] = jnp.zeros(
                (LANES,), jnp.float32)
        return carry

    lax.fori_loop(0, CHUNK, zrow, 0)

    def zcopy(kz, carry):
        pltpu.sync_copy(rows_a, acc_sh.at[pl.ds(s * RPT + kz * CHUNK,
                                                CHUNK)])
        return carry

    lax.fori_loop(0, RPT // CHUNK, zcopy, 0)
    plsc.subcore_barrier()

    def scale_chunk(rows_ref, jj):
        for v in range(CHUNK // LANES):
            idx16 = row_v[jj, pl.ds(v * LANES, LANES)]
            d16 = plsc.load_gather(dis_v, [idx16])
            g_v[pl.ds(v * LANES, LANES)] = (
                d16 * ew_v[jj, pl.ds(v * LANES, LANES)])

        def grp(gi, carry):
            lanes = lax.iota(jnp.int32, LANES)
            e16 = gi * LANES + lanes
            g16 = g_v[pl.ds(gi * LANES, LANES)]
            for f in range(D):
                fo = (lanes + f) & (D - 1)
                val = plsc.load_gather(rows_ref, [e16, fo])
                plsc.store_scatter(rows_ref, [e16, fo], val * g16)
            return carry

        lax.fori_loop(0, CHUNK // LANES, grp, 0)

    def block(bb, carry):
        base = w * CPT + bb * BB
        pltpu.sync_copy(row_hbm.at[pl.ds(base, BB)], row_v)
        pltpu.sync_copy(col_hbm.at[pl.ds(base, BB)], col_v)
        pltpu.sync_copy(ew_hbm.at[pl.ds(base, BB)], ew_v)
        pltpu.async_copy(h_hbm.at[row_v.at[0]], rows_a, gsem_a)
        pltpu.async_copy(h_hbm.at[row_v.at[1]], rows_b, gsem_b)

        def pair(k, carry1):
            ja = 2 * k
            jb = 2 * k + 1

            pltpu.make_async_copy(h_hbm.at[row_v.at[ja]], rows_a,
                                  gsem_a).wait()
            scale_chunk(rows_a, ja)
            pltpu.sync_copy(rows_a, acc_sh.at[col_v.at[ja]], add=True)

            @pl.when(k < BB // 2 - 1)
            def _():
                pltpu.async_copy(h_hbm.at[row_v.at[ja + 2]], rows_a,
                                 gsem_a)

            pltpu.make_async_copy(h_hbm.at[row_v.at[jb]], rows_b,
                                  gsem_b).wait()
            scale_chunk(rows_b, jb)
            pltpu.sync_copy(rows_b, acc_sh.at[col_v.at[jb]], add=True)

            @pl.when(k < BB // 2 - 1)
            def _():
                pltpu.async_copy(h_hbm.at[row_v.at[jb + 2]], rows_b,
                                 gsem_b)

            return carry1

        lax.fori_loop(0, BB // 2, pair, 0)
        return carry

    lax.fori_loop(0, CPT // BB, block, 0)
    plsc.subcore_barrier()
    pltpu.sync_copy(
        acc_sh.at[pl.ds(s * RPT, RPT)],
        accp_hbm.at[c, pl.ds(s * RPT, RPT)],
    )


_msg_call = pl.kernel(
    _msg_body,
    out_type=jax.ShapeDtypeStruct((NC, NPAD, D), jnp.float32),
    mesh=_sc_mesh,
    compiler_params=_sc_params,
    scratch_types=[
        pltpu.VMEM((BB, CHUNK), jnp.int32),
        pltpu.VMEM((BB, CHUNK), jnp.int32),
        pltpu.VMEM((BB, CHUNK), jnp.float32),
        pltpu.VMEM((NPAD,), jnp.float32),
        pltpu.VMEM((CHUNK,), jnp.float32),
        pltpu.VMEM((CHUNK, D), jnp.float32),
        pltpu.VMEM((CHUNK, D), jnp.float32),
        pltpu.VMEM_SHARED((NPAD, D), jnp.float32),
        pltpu.SemaphoreType.DMA,
        pltpu.SemaphoreType.DMA,
    ],
)


MM_BLK = 1280


def _mm_body(x_ref, w_ref, o_ref):
    o_ref[...] = jnp.dot(x_ref[...], w_ref[...],
                         preferred_element_type=jnp.float32)


_mm_call = pl.pallas_call(
    _mm_body,
    grid=(NPAD // MM_BLK,),
    in_specs=[
        pl.BlockSpec((MM_BLK, D), lambda i: (i, 0)),
        pl.BlockSpec((D, D), lambda i: (0, 0)),
    ],
    out_specs=pl.BlockSpec((MM_BLK, D), lambda i: (i, 0)),
    out_shape=jax.ShapeDtypeStruct((NPAD, D), jnp.float32),
)


def _dis_body(degp_ref, dis_ref):
    deg = degp_ref[0, :] + degp_ref[1, :]
    safe = jnp.where(deg > 0, deg, 1.0)
    dis_ref[...] = jnp.where(deg > 0, lax.rsqrt(safe), 0.0)


_dis_call = pl.pallas_call(
    _dis_body,
    out_shape=jax.ShapeDtypeStruct((NPAD,), jnp.float32),
)


FIN_BLK = 1280


def _fin_body(accp_ref, dis_ref, b_ref, o_ref):
    acc = accp_ref[0] + accp_ref[1]
    o = acc * dis_ref[...] + b_ref[...]
    m = jnp.max(o, axis=1, keepdims=True)
    e = jnp.exp(o - m)
    o_ref[...] = e / jnp.sum(e, axis=1, keepdims=True)


_fin_call = pl.pallas_call(
    _fin_body,
    grid=(NPAD // FIN_BLK,),
    in_specs=[
        pl.BlockSpec((NC, FIN_BLK, D), lambda i: (0, i, 0)),
        pl.BlockSpec((FIN_BLK, 1), lambda i: (i, 0)),
        pl.BlockSpec((1, D), lambda i: (0, 0)),
    ],
    out_specs=pl.BlockSpec((FIN_BLK, D), lambda i: (i, 0)),
    out_shape=jax.ShapeDtypeStruct((NPAD, D), jnp.float32),
)


def kernel(x, edge_index, edge_weight, W, b):
    row = edge_index[0].astype(jnp.int32)
    col = edge_index[1].astype(jnp.int32)
    pad = EP - E
    rowp = jnp.concatenate(
        [row, jnp.full((pad,), PAD_IDX, jnp.int32)]).reshape(EP // CHUNK,
                                                             CHUNK)
    colp = jnp.concatenate(
        [col, jnp.full((pad,), PAD_IDX, jnp.int32)]).reshape(EP // CHUNK,
                                                             CHUNK)
    ewp = jnp.concatenate(
        [edge_weight.astype(jnp.float32),
         jnp.zeros((pad,), jnp.float32)]).reshape(EP // CHUNK, CHUNK)
    xp = jnp.concatenate(
        [x.astype(jnp.float32), jnp.zeros((NPAD - N, D), jnp.float32)])

    degp = _deg_call(colp, ewp)
    h = _mm_call(xp, W.astype(jnp.float32))
    dis = _dis_call(degp)
    accp = _msg_call(rowp, colp, ewp, dis, h)
    out = _fin_call(accp, dis.reshape(NPAD, 1),
                    b.astype(jnp.float32).reshape(1, D))
    return out[:N]
